# baseline (device time: 47387 ns/iter reference)
import jax
import jax.numpy as jnp
from jax import lax
from jax.experimental import pallas as pl
from jax.experimental.pallas import tpu as pltpu

N_DEV = 32
SQ = 256
D = 1024
DH = 128
HQ_LOCAL = 8
GROUP = 4
HKV_LOCAL = HQ_LOCAL // GROUP
ROWS = SQ // N_DEV
SCALE = 0.08838834764831843


def kernel(x, Wq, Wo, Wk, Wv):
    my = lax.axis_index("i")
    x2 = x.reshape(SQ, D)
    kv_cols = HKV_LOCAL * DH
    Wk_sl = lax.dynamic_slice_in_dim(Wk, my * kv_cols, kv_cols, axis=1)
    Wv_sl = lax.dynamic_slice_in_dim(Wv, my * kv_cols, kv_cols, axis=1)

    def body(x_ref, wq_ref, wo_ref, wk_ref, wv_ref, out_ref,
             partial_ref, rs_ref, red_ref, ag_ref,
             send1, recv1, send2, recv2):
        my_pos = lax.axis_index("i")

        barrier = pltpu.get_barrier_semaphore()
        for dj in range(1, N_DEV):
            tgt = lax.rem(my_pos + dj, N_DEV)
            pl.semaphore_signal(barrier, inc=1, device_id=(tgt,),
                                device_id_type=pl.DeviceIdType.MESH)
        pl.semaphore_wait(barrier, N_DEV - 1)

        xb = x_ref[...].astype(jnp.bfloat16)
        q = jnp.dot(xb, wq_ref[...].astype(jnp.bfloat16),
                    preferred_element_type=jnp.float32).astype(jnp.bfloat16)
        k = jnp.dot(xb, wk_ref[...].astype(jnp.bfloat16),
                    preferred_element_type=jnp.float32).astype(jnp.bfloat16)
        v = jnp.dot(xb, wv_ref[...].astype(jnp.bfloat16),
                    preferred_element_type=jnp.float32).astype(jnp.bfloat16)

        heads = []
        for h in range(HQ_LOCAL):
            qh = q[:, h * DH:(h + 1) * DH]
            g = h // GROUP
            kh = k[:, g * DH:(g + 1) * DH]
            vh = v[:, g * DH:(g + 1) * DH]
            s = lax.dot_general(qh, kh, (((1,), (1,)), ((), ())),
                                preferred_element_type=jnp.float32) * SCALE
            m = jnp.max(s, axis=1, keepdims=True)
            p = jnp.exp(s - m)
            l = jnp.sum(p, axis=1, keepdims=True)
            o = lax.dot_general(p.astype(jnp.bfloat16), vh,
                                (((1,), (0,)), ((), ())),
                                preferred_element_type=jnp.float32)
            heads.append((o / l).astype(jnp.bfloat16))
        attn = jnp.concatenate(heads, axis=1)

        partial_ref[...] = jnp.dot(attn, wo_ref[...].astype(jnp.bfloat16),
                                   preferred_element_type=jnp.float32)

        rdmas1 = []
        for dj in range(1, N_DEV):
            tgt = lax.rem(my_pos + dj, N_DEV)
            r = pltpu.make_async_remote_copy(
                src_ref=partial_ref.at[pl.ds(tgt * ROWS, ROWS), :],
                dst_ref=rs_ref.at[dj - 1],
                send_sem=send1.at[dj],
                recv_sem=recv1.at[dj],
                device_id=(tgt,),
                device_id_type=pl.DeviceIdType.MESH,
            )
            r.start()
            rdmas1.append(r)

        acc = partial_ref[pl.ds(my_pos * ROWS, ROWS), :]
        for r in rdmas1:
            r.wait_recv()
        for j in range(N_DEV - 1):
            acc = acc + rs_ref[j]
        red_ref[...] = acc

        rdmas2 = []
        for dj in range(1, N_DEV):
            tgt = lax.rem(my_pos + dj, N_DEV)
            r = pltpu.make_async_remote_copy(
                src_ref=red_ref,
                dst_ref=ag_ref.at[pl.ds(my_pos * ROWS, ROWS), :],
                send_sem=send2.at[dj],
                recv_sem=recv2.at[dj],
                device_id=(tgt,),
                device_id_type=pl.DeviceIdType.MESH,
            )
            r.start()
            rdmas2.append(r)

        ag_ref[pl.ds(my_pos * ROWS, ROWS), :] = acc
        for r in rdmas2:
            r.wait_recv()
        out_ref[...] = ag_ref[...]

        for r in rdmas1:
            r.wait_send()
        for r in rdmas2:
            r.wait_send()

    out = pl.pallas_call(
        body,
        out_shape=jax.ShapeDtypeStruct((SQ, D), jnp.float32),
        in_specs=[pl.BlockSpec(memory_space=pltpu.VMEM)] * 5,
        out_specs=pl.BlockSpec(memory_space=pltpu.VMEM),
        scratch_shapes=[
            pltpu.VMEM((SQ, D), jnp.float32),
            pltpu.VMEM((N_DEV - 1, ROWS, D), jnp.float32),
            pltpu.VMEM((ROWS, D), jnp.float32),
            pltpu.VMEM((SQ, D), jnp.float32),
            pltpu.SemaphoreType.DMA((N_DEV,)),
            pltpu.SemaphoreType.DMA((N_DEV,)),
            pltpu.SemaphoreType.DMA((N_DEV,)),
            pltpu.SemaphoreType.DMA((N_DEV,)),
        ],
        compiler_params=pltpu.CompilerParams(collective_id=0),
    )(x2, Wq, Wo, Wk_sl, Wv_sl)
    return out.reshape(1, SQ, D)


# device time: 33648 ns/iter; 1.4083x vs baseline; 1.4083x over previous
import jax
import jax.numpy as jnp
from jax import lax
from jax.experimental import pallas as pl
from jax.experimental.pallas import tpu as pltpu

N_DEV = 32
SQ = 256
D = 1024
DH = 128
HQ_LOCAL = 8
GROUP = 4
HKV_LOCAL = HQ_LOCAL // GROUP
ROWS = SQ // N_DEV
SCALE = 0.08838834764831843


def kernel(x, Wq, Wo, Wk, Wv):
    my = lax.axis_index("i")
    x2 = x.reshape(SQ, D)
    kv_cols = HKV_LOCAL * DH
    Wk_sl = lax.dynamic_slice_in_dim(Wk, my * kv_cols, kv_cols, axis=1)
    Wv_sl = lax.dynamic_slice_in_dim(Wv, my * kv_cols, kv_cols, axis=1)

    def body(x_ref, wq_ref, wo_ref, wk_ref, wv_ref, out_ref,
             partial_ref, p16_ref, rs_ref, red_ref, ag_ref,
             send1, recv1, send2, recv2):
        my_pos = lax.axis_index("i")

        barrier = pltpu.get_barrier_semaphore()
        for dj in range(1, N_DEV):
            tgt = lax.rem(my_pos + dj, N_DEV)
            pl.semaphore_signal(barrier, inc=1, device_id=(tgt,),
                                device_id_type=pl.DeviceIdType.MESH)

        xb = x_ref[...].astype(jnp.bfloat16)
        q = jnp.dot(xb, wq_ref[...].astype(jnp.bfloat16),
                    preferred_element_type=jnp.float32).astype(jnp.bfloat16)
        k = jnp.dot(xb, wk_ref[...].astype(jnp.bfloat16),
                    preferred_element_type=jnp.float32).astype(jnp.bfloat16)
        v = jnp.dot(xb, wv_ref[...].astype(jnp.bfloat16),
                    preferred_element_type=jnp.float32).astype(jnp.bfloat16)

        heads = []
        for h in range(HQ_LOCAL):
            qh = q[:, h * DH:(h + 1) * DH]
            g = h // GROUP
            kh = k[:, g * DH:(g + 1) * DH]
            vh = v[:, g * DH:(g + 1) * DH]
            s = lax.dot_general(qh, kh, (((1,), (1,)), ((), ())),
                                preferred_element_type=jnp.float32) * SCALE
            m = jnp.max(s, axis=1, keepdims=True)
            p = jnp.exp(s - m)
            l = jnp.sum(p, axis=1, keepdims=True)
            o = lax.dot_general(p.astype(jnp.bfloat16), vh,
                                (((1,), (0,)), ((), ())),
                                preferred_element_type=jnp.float32)
            heads.append((o / l).astype(jnp.bfloat16))
        attn = jnp.concatenate(heads, axis=1)

        partial = jnp.dot(attn, wo_ref[...].astype(jnp.bfloat16),
                          preferred_element_type=jnp.float32)
        partial_ref[...] = partial
        p16_ref[...] = partial.astype(jnp.bfloat16)

        pl.semaphore_wait(barrier, N_DEV - 1)

        rdmas1 = []
        for dj in range(1, N_DEV):
            tgt = lax.rem(my_pos + dj, N_DEV)
            r = pltpu.make_async_remote_copy(
                src_ref=p16_ref.at[pl.ds(tgt * ROWS, ROWS), :],
                dst_ref=rs_ref.at[dj - 1],
                send_sem=send1.at[dj],
                recv_sem=recv1.at[dj],
                device_id=(tgt,),
                device_id_type=pl.DeviceIdType.MESH,
            )
            r.start()
            rdmas1.append(r)

        acc = partial_ref[pl.ds(my_pos * ROWS, ROWS), :]
        for r in rdmas1:
            r.wait_recv()
        for j in range(N_DEV - 1):
            acc = acc + rs_ref[j].astype(jnp.float32)
        red_ref[...] = acc.astype(jnp.bfloat16)

        rdmas2 = []
        for dj in range(1, N_DEV):
            tgt = lax.rem(my_pos + dj, N_DEV)
            r = pltpu.make_async_remote_copy(
                src_ref=red_ref,
                dst_ref=ag_ref.at[pl.ds(my_pos * ROWS, ROWS), :],
                send_sem=send2.at[dj],
                recv_sem=recv2.at[dj],
                device_id=(tgt,),
                device_id_type=pl.DeviceIdType.MESH,
            )
            r.start()
            rdmas2.append(r)

        ag_ref[pl.ds(my_pos * ROWS, ROWS), :] = acc.astype(jnp.bfloat16)
        for r in rdmas2:
            r.wait_recv()
        out_ref[...] = ag_ref[...].astype(jnp.float32)

        for r in rdmas1:
            r.wait_send()
        for r in rdmas2:
            r.wait_send()

    out = pl.pallas_call(
        body,
        out_shape=jax.ShapeDtypeStruct((SQ, D), jnp.float32),
        in_specs=[pl.BlockSpec(memory_space=pltpu.VMEM)] * 5,
        out_specs=pl.BlockSpec(memory_space=pltpu.VMEM),
        scratch_shapes=[
            pltpu.VMEM((SQ, D), jnp.float32),
            pltpu.VMEM((SQ, D), jnp.bfloat16),
            pltpu.VMEM((N_DEV - 1, ROWS, D), jnp.bfloat16),
            pltpu.VMEM((ROWS, D), jnp.bfloat16),
            pltpu.VMEM((SQ, D), jnp.bfloat16),
            pltpu.SemaphoreType.DMA((N_DEV,)),
            pltpu.SemaphoreType.DMA((N_DEV,)),
            pltpu.SemaphoreType.DMA((N_DEV,)),
            pltpu.SemaphoreType.DMA((N_DEV,)),
        ],
        compiler_params=pltpu.CompilerParams(collective_id=0),
    )(x2, Wq, Wo, Wk_sl, Wv_sl)
    return out.reshape(1, SQ, D)
